# baseline (speedup 1.0000x reference)
import functools

import jax
import jax.numpy as jnp
from jax.experimental import pallas as pl
from jax.experimental.pallas import tpu as pltpu


def _outer_kernel(x_hbm, w1_ref, b1_ref, w2_ref, b2_ref, mu_hbm,
                  *, tb, n_tiles, bufs):
    def body(x_blk, mu_blk):
        xb = x_blk[...].astype(jnp.bfloat16)
        w1b = w1_ref[...].astype(jnp.bfloat16)
        h = jnp.dot(xb, w1b, preferred_element_type=jnp.float32)
        h = jnp.maximum(h + b1_ref[...], 0.0)
        w2b = w2_ref[...].astype(jnp.bfloat16)
        mu = jnp.dot(h.astype(jnp.bfloat16), w2b,
                     preferred_element_type=jnp.float32)
        mu_blk[...] = mu + b2_ref[...]

    S = x_hbm.shape[1]
    A = mu_hbm.shape[1]
    pipe = pltpu.emit_pipeline(
        body,
        grid=(n_tiles,),
        in_specs=[pl.BlockSpec(
            (tb, S), lambda i: (i, 0),
            pipeline_mode=pl.Buffered(buffer_count=bufs,
                                      use_lookahead=True))],
        out_specs=[pl.BlockSpec(
            (tb, A), lambda i: (i, 0),
            pipeline_mode=pl.Buffered(buffer_count=2))],
    )
    pipe(x_hbm, mu_hbm)


@functools.partial(jax.jit, static_argnames=("tb", "bufs"))
def _forward(x, w1, b1, w2, b2, tb=4096, bufs=6):
    B, S = x.shape
    H = w1.shape[1]
    A = w2.shape[1]
    n_tiles = B // tb
    return pl.pallas_call(
        functools.partial(_outer_kernel, tb=tb, n_tiles=n_tiles, bufs=bufs),
        out_shape=jax.ShapeDtypeStruct((B, A), jnp.float32),
        in_specs=[
            pl.BlockSpec(memory_space=pl.ANY),
            pl.BlockSpec(memory_space=pltpu.MemorySpace.VMEM),
            pl.BlockSpec(memory_space=pltpu.MemorySpace.VMEM),
            pl.BlockSpec(memory_space=pltpu.MemorySpace.VMEM),
            pl.BlockSpec(memory_space=pltpu.MemorySpace.VMEM),
        ],
        out_specs=pl.BlockSpec(memory_space=pl.ANY),
    )(x, w1, b1, w2, b2)


def kernel(x, w1, b1, w2, b2, sigma_param, episode_number):
    mu = _forward(x, w1, b1, w2, b2)
    sigma = jnp.asarray(5.0, dtype=jnp.float32)
    return mu, sigma


# speedup vs baseline: 1.0412x; 1.0412x over previous
import functools

import jax
import jax.numpy as jnp
from jax.experimental import pallas as pl
from jax.experimental.pallas import tpu as pltpu


def _outer_kernel(x_hbm, w1_ref, b1_ref, w2_ref, b2_ref, mu_hbm,
                  *, tb, n_tiles, bufs):
    def body(x_blk, mu_blk):
        xb = x_blk[...].astype(jnp.bfloat16)
        w1b = w1_ref[...].astype(jnp.bfloat16)
        h = jnp.dot(xb, w1b, preferred_element_type=jnp.float32)
        h = jnp.maximum(h + b1_ref[...], 0.0)
        w2b = w2_ref[...].astype(jnp.bfloat16)
        mu = jnp.dot(h.astype(jnp.bfloat16), w2b,
                     preferred_element_type=jnp.float32)
        mu_blk[...] = mu + b2_ref[...]

    S = x_hbm.shape[1]
    A = mu_hbm.shape[1]
    pipe = pltpu.emit_pipeline(
        body,
        grid=(n_tiles,),
        in_specs=[pl.BlockSpec(
            (tb, S), lambda i: (i, 0),
            pipeline_mode=pl.Buffered(buffer_count=bufs,
                                      use_lookahead=True))],
        out_specs=[pl.BlockSpec(
            (tb, A), lambda i: (i, 0),
            pipeline_mode=pl.Buffered(buffer_count=2))],
    )
    pipe(x_hbm, mu_hbm)


@functools.partial(jax.jit, static_argnames=("tb", "bufs"))
def _forward(x, w1, b1, w2, b2, tb=8192, bufs=4):
    B, S = x.shape
    H = w1.shape[1]
    A = w2.shape[1]
    n_tiles = B // tb
    return pl.pallas_call(
        functools.partial(_outer_kernel, tb=tb, n_tiles=n_tiles, bufs=bufs),
        out_shape=jax.ShapeDtypeStruct((B, A), jnp.float32),
        in_specs=[
            pl.BlockSpec(memory_space=pl.ANY),
            pl.BlockSpec(memory_space=pltpu.MemorySpace.VMEM),
            pl.BlockSpec(memory_space=pltpu.MemorySpace.VMEM),
            pl.BlockSpec(memory_space=pltpu.MemorySpace.VMEM),
            pl.BlockSpec(memory_space=pltpu.MemorySpace.VMEM),
        ],
        out_specs=pl.BlockSpec(memory_space=pl.ANY),
    )(x, w1, b1, w2, b2)


def kernel(x, w1, b1, w2, b2, sigma_param, episode_number):
    mu = _forward(x, w1, b1, w2, b2)
    sigma = jnp.asarray(5.0, dtype=jnp.float32)
    return mu, sigma
